# baseline (speedup 1.0000x reference)
import functools

import jax
import jax.numpy as jnp
from jax import lax
from jax.experimental import pallas as pl
from jax.experimental.pallas import tpu as pltpu
from jax.experimental.pallas import tpu_sc as plsc

_NUM_CORES = 2
_NUM_SUBCORES = 16
_NUM_WORKERS = _NUM_CORES * _NUM_SUBCORES
_LANES = 16
_SC_SHARE = 8192


def _gather_sc(data, idx_sc):
    num_rows, row_dim = data.shape
    batch = idx_sc.shape[0]
    b_per_w = batch // _NUM_WORKERS
    mesh = plsc.VectorSubcoreMesh(core_axis_name="c", subcore_axis_name="s")

    @functools.partial(
        pl.kernel,
        mesh=mesh,
        out_type=jax.ShapeDtypeStruct((batch, row_dim), data.dtype),
        scratch_types=[
            pltpu.VMEM((b_per_w,), jnp.int32),
            pltpu.VMEM((b_per_w, row_dim), jnp.float32),
            pltpu.SemaphoreType.DMA,
        ],
    )
    def gather_rows_sc(table_hbm, idx_hbm, out_hbm, idx_v, rows_v, sem):
        wid = lax.axis_index("s") * _NUM_CORES + lax.axis_index("c")
        base = wid * b_per_w
        pltpu.sync_copy(idx_hbm.at[pl.ds(base, b_per_w)], idx_v)

        @pl.loop(0, b_per_w, step=_LANES)
        def _(g):
            vec = idx_v[pl.ds(g, _LANES)]
            for j in range(_LANES):
                pltpu.async_copy(
                    table_hbm.at[pl.ds(vec[j], 1)],
                    rows_v.at[pl.ds(g + j, 1)],
                    sem,
                )

        pltpu.make_async_copy(
            table_hbm.at[pl.ds(0, b_per_w)], rows_v, sem
        ).wait()

        pltpu.sync_copy(rows_v, out_hbm.at[pl.ds(base, b_per_w)])

    return gather_rows_sc(data, idx_sc)


def _gather_tc(data, idx_tc):
    num_rows, row_dim = data.shape
    batch = idx_tc.shape[0]

    def body(idx_smem, table_hbm, out_hbm, sem):
        @pl.loop(0, batch)
        def _(i):
            pltpu.make_async_copy(
                table_hbm.at[pl.ds(idx_smem[i], 1)],
                out_hbm.at[pl.ds(i, 1)],
                sem,
            ).start()

        pltpu.make_async_copy(
            table_hbm.at[pl.ds(0, batch)], out_hbm, sem
        ).wait()

    return pl.pallas_call(
        body,
        in_specs=[
            pl.BlockSpec(memory_space=pltpu.MemorySpace.SMEM),
            pl.BlockSpec(memory_space=pltpu.MemorySpace.HBM),
        ],
        out_specs=pl.BlockSpec(memory_space=pltpu.MemorySpace.HBM),
        out_shape=jax.ShapeDtypeStruct((batch, row_dim), data.dtype),
        scratch_shapes=[pltpu.SemaphoreType.DMA],
    )(idx_tc, data)


def kernel(data, indices):
    batch = indices.shape[0]
    idx = indices.reshape(batch).astype(jnp.int32)
    out_sc = _gather_sc(data, idx[:_SC_SHARE])
    out_tc = _gather_tc(data, idx[_SC_SHARE:])
    return jnp.concatenate([out_sc, out_tc], axis=0)


# speedup vs baseline: 1.3588x; 1.3588x over previous
import functools

import jax
import jax.numpy as jnp
from jax import lax
from jax.experimental import pallas as pl
from jax.experimental.pallas import tpu as pltpu
from jax.experimental.pallas import tpu_sc as plsc

_NUM_CORES = 2
_NUM_SUBCORES = 16
_NUM_WORKERS = _NUM_CORES * _NUM_SUBCORES
_LANES = 16


def kernel(data, indices):
    num_rows, row_dim = data.shape
    batch = indices.shape[0]
    idx = indices.reshape(batch).astype(jnp.int32)
    b_per_w = batch // _NUM_WORKERS

    mesh = plsc.VectorSubcoreMesh(core_axis_name="c", subcore_axis_name="s")

    @functools.partial(
        pl.kernel,
        mesh=mesh,
        out_type=jax.ShapeDtypeStruct((batch, row_dim), data.dtype),
        scratch_types=[
            pltpu.VMEM((b_per_w,), jnp.int32),
            pltpu.VMEM((b_per_w, row_dim), jnp.float32),
            pltpu.SemaphoreType.DMA,
        ],
    )
    def gather_rows_sc(table_hbm, idx_hbm, out_hbm, idx_v, rows_v, sem):
        wid = lax.axis_index("s") * _NUM_CORES + lax.axis_index("c")
        base = wid * b_per_w
        pltpu.sync_copy(idx_hbm.at[pl.ds(base, b_per_w)], idx_v)

        @pl.loop(0, b_per_w, step=_LANES)
        def _(g):
            vec = idx_v[pl.ds(g, _LANES)]
            for j in range(_LANES):
                pltpu.async_copy(
                    table_hbm.at[pl.ds(vec[j], 1)],
                    rows_v.at[pl.ds(g + j, 1)],
                    sem,
                )

        pltpu.make_async_copy(
            table_hbm.at[pl.ds(0, b_per_w)],
            rows_v,
            sem,
        ).wait()

        pltpu.sync_copy(rows_v, out_hbm.at[pl.ds(base, b_per_w)])

    return gather_rows_sc(data, idx)
